# baseline (speedup 1.0000x reference)
import jax
import jax.numpy as jnp
from jax.experimental import pallas as pl
from jax.experimental.pallas import tpu as pltpu

_K = 1024
_D = 64
_BETA = 0.25
_BLK = 512


def _vq_body(z_ref, zsqt_ref, cb_ref, esq_ref,
             zq_ref, idx_ref, loss_ref, perp_ref,
             loss_acc, cnt_acc, *, n_rows, grid):
    i = pl.program_id(0)
    z = z_ref[...]
    dots = jax.lax.dot_general(
        cb_ref[...], z, (((1,), (1,)), ((), ())),
        preferred_element_type=jnp.float32)
    d = (zsqt_ref[...] + esq_ref[...]) - 2.0 * dots
    mind = jnp.min(d, axis=0, keepdims=True)
    iota = jax.lax.broadcasted_iota(jnp.int32, (_K, _BLK), 0)
    idx = jnp.min(jnp.where(d == mind, iota, _K),
                  axis=0, keepdims=True)
    idx_ref[...] = idx.reshape(1, 1, _BLK)
    onehot = (iota == idx).astype(jnp.float32)
    zq = jax.lax.dot_general(
        onehot, cb_ref[...], (((0,), (0,)), ((), ())),
        preferred_element_type=jnp.float32)
    zq_ref[...] = zq

    @pl.when(i == 0)
    def _init():
        loss_acc[...] = jnp.zeros_like(loss_acc)
        cnt_acc[...] = jnp.zeros_like(cnt_acc)

    loss_acc[...] += jnp.sum(mind, axis=(0, 1), keepdims=True)
    part = onehot[:, 0:128]
    for c in range(128, _BLK, 128):
        part = part + onehot[:, c:c + 128]
    cnt_acc[...] += part

    @pl.when(i == grid - 1)
    def _fini():
        loss_ref[...] = (1.0 + _BETA) * loss_acc[...] / (n_rows * _D)
        cnt = jnp.sum(cnt_acc[...], axis=1, keepdims=True)
        avg = cnt / n_rows
        ent = jnp.sum(avg * jnp.log(avg + 1e-12), axis=(0, 1), keepdims=True)
        perp_ref[...] = jnp.exp(-ent)


def kernel(z_e, codebook):
    z = z_e.reshape(-1, _D)
    n_rows = z.shape[0]
    grid = n_rows // _BLK
    zsqt = jnp.sum(z ** 2, axis=1).reshape(1, n_rows)
    esq = jnp.sum(codebook ** 2, axis=1).reshape(_K, 1)

    body = lambda *refs: _vq_body(*refs, n_rows=n_rows, grid=grid)
    zq, idx, loss, perp = pl.pallas_call(
        body,
        grid=(grid,),
        in_specs=[
            pl.BlockSpec((_BLK, _D), lambda i: (i, 0)),
            pl.BlockSpec((1, _BLK), lambda i: (0, i)),
            pl.BlockSpec((_K, _D), lambda i: (0, 0)),
            pl.BlockSpec((_K, 1), lambda i: (0, 0)),
        ],
        out_specs=[
            pl.BlockSpec((_BLK, _D), lambda i: (i, 0)),
            pl.BlockSpec((1, 1, _BLK), lambda i: (i, 0, 0)),
            pl.BlockSpec((1, 1), lambda i: (0, 0)),
            pl.BlockSpec((1, 1), lambda i: (0, 0)),
        ],
        out_shape=[
            jax.ShapeDtypeStruct((n_rows, _D), jnp.float32),
            jax.ShapeDtypeStruct((grid, 1, _BLK), jnp.int32),
            jax.ShapeDtypeStruct((1, 1), jnp.float32),
            jax.ShapeDtypeStruct((1, 1), jnp.float32),
        ],
        scratch_shapes=[
            pltpu.VMEM((1, 1), jnp.float32),
            pltpu.VMEM((_K, 128), jnp.float32),
        ],
    )(z, zsqt, codebook, esq)

    z_q = zq.reshape(z_e.shape)
    indices = idx.reshape(z_e.shape[:-1])
    return (z_q, indices, loss[0, 0], perp[0, 0])


# speedup vs baseline: 1.1375x; 1.1375x over previous
import jax
import jax.numpy as jnp
from jax.experimental import pallas as pl
from jax.experimental.pallas import tpu as pltpu

_K = 1024
_D = 64
_BETA = 0.25
_BB = 8
_TB = 128
_BLK = _BB * _TB


def _vq_body(z_ref, zsq_ref, cb_ref, esq_ref,
             zq_ref, idx_ref, loss_ref, perp_ref,
             loss_acc, cnt_acc, *, n_rows, gb, gt):
    b = pl.program_id(0)
    t = pl.program_id(1)
    z = z_ref[...].reshape(_BLK, _D)
    zsq8 = zsq_ref[...]
    zsqt = jnp.concatenate(
        [zsq8[r:r + 1, :] for r in range(_BB)], axis=1)
    dots = jax.lax.dot_general(
        cb_ref[...], z, (((1,), (1,)), ((), ())),
        preferred_element_type=jnp.float32)
    d = (zsqt + esq_ref[...]) - 2.0 * dots
    mind = jnp.min(d, axis=0, keepdims=True)
    iota = jax.lax.broadcasted_iota(jnp.int32, (_K, _BLK), 0)
    idx = jnp.min(jnp.where(d == mind, iota, _K),
                  axis=0, keepdims=True)
    idx_ref[...] = jnp.concatenate(
        [idx[:, c * _TB:(c + 1) * _TB] for c in range(_BB)], axis=0)
    onehot = (iota == idx).astype(jnp.float32)
    zq = jax.lax.dot_general(
        onehot, cb_ref[...], (((0,), (0,)), ((), ())),
        preferred_element_type=jnp.float32)
    zq_ref[...] = zq.reshape(_BB, _TB, _D)

    @pl.when((b == 0) & (t == 0))
    def _init():
        loss_acc[...] = jnp.zeros_like(loss_acc)
        cnt_acc[...] = jnp.zeros_like(cnt_acc)

    loss_acc[...] += jnp.sum(mind, axis=(0, 1), keepdims=True)
    part = onehot[:, 0:128]
    for c in range(128, _BLK, 128):
        part = part + onehot[:, c:c + 128]
    cnt_acc[...] += part

    @pl.when((b == gb - 1) & (t == gt - 1))
    def _fini():
        loss_ref[...] = (1.0 + _BETA) * loss_acc[...] / (n_rows * _D)
        cnt = jnp.sum(cnt_acc[...], axis=1, keepdims=True)
        avg = cnt / n_rows
        ent = jnp.sum(avg * jnp.log(avg + 1e-12), axis=(0, 1), keepdims=True)
        perp_ref[...] = jnp.exp(-ent)


def kernel(z_e, codebook):
    nb, nt, _ = z_e.shape
    n_rows = nb * nt
    gb, gt = nb // _BB, nt // _TB
    zsq = jnp.sum(z_e ** 2, axis=2)
    esq = jnp.sum(codebook ** 2, axis=1).reshape(_K, 1)

    body = lambda *refs: _vq_body(*refs, n_rows=n_rows, gb=gb, gt=gt)
    zq, idx, loss, perp = pl.pallas_call(
        body,
        grid=(gb, gt),
        in_specs=[
            pl.BlockSpec((_BB, _TB, _D), lambda b, t: (b, t, 0)),
            pl.BlockSpec((_BB, _TB), lambda b, t: (b, t)),
            pl.BlockSpec((_K, _D), lambda b, t: (0, 0)),
            pl.BlockSpec((_K, 1), lambda b, t: (0, 0)),
        ],
        out_specs=[
            pl.BlockSpec((_BB, _TB, _D), lambda b, t: (b, t, 0)),
            pl.BlockSpec((_BB, _TB), lambda b, t: (b, t)),
            pl.BlockSpec((1, 1), lambda b, t: (0, 0)),
            pl.BlockSpec((1, 1), lambda b, t: (0, 0)),
        ],
        out_shape=[
            jax.ShapeDtypeStruct((nb, nt, _D), jnp.float32),
            jax.ShapeDtypeStruct((nb, nt), jnp.int32),
            jax.ShapeDtypeStruct((1, 1), jnp.float32),
            jax.ShapeDtypeStruct((1, 1), jnp.float32),
        ],
        scratch_shapes=[
            pltpu.VMEM((1, 1), jnp.float32),
            pltpu.VMEM((_K, 128), jnp.float32),
        ],
    )(z_e, zsq, codebook, esq)

    return (zq, idx, loss[0, 0], perp[0, 0])


# speedup vs baseline: 1.3200x; 1.1604x over previous
import jax
import jax.numpy as jnp
from jax.experimental import pallas as pl
from jax.experimental.pallas import tpu as pltpu

_K = 1024
_D = 64
_BETA = 0.25
_BB = 8
_TB = 256
_BLK = _BB * _TB


def _vq_body(z_ref, zsq_ref, cb_ref, esq_ref,
             zq_ref, idx_ref, loss_ref, perp_ref,
             loss_acc, cnt_acc, *, n_rows, gb, gt):
    b = pl.program_id(0)
    t = pl.program_id(1)
    z = z_ref[...].reshape(_BLK, _D)
    zsq8 = zsq_ref[...]
    zsqt = zsq8.reshape(1, _BLK)
    dots = jax.lax.dot_general(
        cb_ref[...], z, (((1,), (1,)), ((), ())),
        preferred_element_type=jnp.float32)
    d = (zsqt + esq_ref[...]) - 2.0 * dots
    mind = jnp.min(d, axis=0, keepdims=True)
    iota = jax.lax.broadcasted_iota(jnp.int32, (_K, _BLK), 0)
    idx = jnp.min(jnp.where(d == mind, iota, _K),
                  axis=0, keepdims=True)
    idx_ref[...] = idx.reshape(_BB, _TB)
    onehot = (iota == idx).astype(jnp.float32)
    zq = jax.lax.dot_general(
        onehot, cb_ref[...], (((0,), (0,)), ((), ())),
        preferred_element_type=jnp.float32)
    zq_ref[...] = zq.reshape(_BB, _TB, _D)

    @pl.when((b == 0) & (t == 0))
    def _init():
        loss_acc[...] = jnp.zeros_like(loss_acc)
        cnt_acc[...] = jnp.zeros_like(cnt_acc)

    loss_acc[...] += jnp.sum(mind, axis=(0, 1), keepdims=True)
    part = onehot[:, 0:128]
    for c in range(128, _BLK, 128):
        part = part + onehot[:, c:c + 128]
    cnt_acc[...] += part

    @pl.when((b == gb - 1) & (t == gt - 1))
    def _fini():
        loss_ref[...] = (1.0 + _BETA) * loss_acc[...] / (n_rows * _D)
        cnt = jnp.sum(cnt_acc[...], axis=1, keepdims=True)
        avg = cnt / n_rows
        ent = jnp.sum(avg * jnp.log(avg + 1e-12), axis=(0, 1), keepdims=True)
        perp_ref[...] = jnp.exp(-ent)


def kernel(z_e, codebook):
    nb, nt, _ = z_e.shape
    n_rows = nb * nt
    gb, gt = nb // _BB, nt // _TB
    zsq = jnp.sum(z_e ** 2, axis=2)
    esq = jnp.sum(codebook ** 2, axis=1).reshape(_K, 1)

    body = lambda *refs: _vq_body(*refs, n_rows=n_rows, gb=gb, gt=gt)
    zq, idx, loss, perp = pl.pallas_call(
        body,
        grid=(gb, gt),
        in_specs=[
            pl.BlockSpec((_BB, _TB, _D), lambda b, t: (b, t, 0)),
            pl.BlockSpec((_BB, _TB), lambda b, t: (b, t)),
            pl.BlockSpec((_K, _D), lambda b, t: (0, 0)),
            pl.BlockSpec((_K, 1), lambda b, t: (0, 0)),
        ],
        out_specs=[
            pl.BlockSpec((_BB, _TB, _D), lambda b, t: (b, t, 0)),
            pl.BlockSpec((_BB, _TB), lambda b, t: (b, t)),
            pl.BlockSpec((1, 1), lambda b, t: (0, 0)),
            pl.BlockSpec((1, 1), lambda b, t: (0, 0)),
        ],
        out_shape=[
            jax.ShapeDtypeStruct((nb, nt, _D), jnp.float32),
            jax.ShapeDtypeStruct((nb, nt), jnp.int32),
            jax.ShapeDtypeStruct((1, 1), jnp.float32),
            jax.ShapeDtypeStruct((1, 1), jnp.float32),
        ],
        scratch_shapes=[
            pltpu.VMEM((1, 1), jnp.float32),
            pltpu.VMEM((_K, 128), jnp.float32),
        ],
    )(z_e, zsq, codebook, esq)

    return (zq, idx, loss[0, 0], perp[0, 0])
